# baseline (speedup 1.0000x reference)
import functools

import jax
import jax.numpy as jnp
from jax import lax
from jax.experimental import pallas as pl
from jax.experimental.pallas import tpu as pltpu
from jax.experimental.pallas import tpu_sc as plsc

VOCAB = 1000000
NTURNS = 4
OUTDIM = 32
DEG = 4
B = 4096
S = 200
N = B * S
NGRAN = VOCAB // 4

_info = plsc.get_sparse_core_info()
NC, NS = _info.num_cores, _info.num_subcores
NW = NC * NS

_sc_params = pltpu.CompilerParams(
    use_tc_tiling_on_sc=False, needs_layout_passes=False
)
_mesh = plsc.VectorSubcoreMesh(core_axis_name="c", subcore_axis_name="s")


_CG = 1956
_ICHUNKS = 4
_GW = _CG * _ICHUNKS
assert _CG * 4 % 16 == 0 and NW * _GW >= NGRAN and _GW <= NGRAN


@functools.partial(
    pl.kernel,
    mesh=_mesh,
    compiler_params=_sc_params,
    out_type=jax.ShapeDtypeStruct((NGRAN, 16), jnp.float32),
    scratch_types=[
        pltpu.VMEM((NTURNS, 4 * _CG), jnp.float32),
        pltpu.VMEM((_CG, 16), jnp.float32),
    ],
)
def _sc_interleave(turnsT_hbm, out_hbm, in_v, out_v):
    wid = lax.axis_index("s") * NC + lax.axis_index("c")
    gw0 = jnp.minimum(wid * _GW, NGRAN - _GW)
    lane = lax.iota(jnp.int32, 16)
    grow = lane >> 2
    gcol = (lane & 3) * 4

    def chunk(i, carry):
        g0 = gw0 + i * _CG
        v0 = g0 * 4
        for t in range(NTURNS):
            pltpu.sync_copy(turnsT_hbm.at[t, pl.ds(v0, 4 * _CG)], in_v.at[t])

        def body(u, c):
            for t in range(NTURNS):
                vals = in_v[t, pl.ds(u * 16, 16)]
                plsc.store_scatter(out_v, [(u * 4) + grow, gcol + t], vals)
            return c

        lax.fori_loop(0, 4 * _CG // 16, body, 0)
        pltpu.sync_copy(out_v, out_hbm.at[pl.ds(g0, _CG)])
        return carry

    lax.fori_loop(0, _ICHUNKS, chunk, 0)


_UNIT = 512
CH = 2560
_UPC = CH // _UNIT
_BPW = N // NW
_NCHUNK = _BPW // CH


@functools.partial(
    pl.kernel,
    mesh=_mesh,
    compiler_params=_sc_params,
    out_type=jax.ShapeDtypeStruct((N // 32, 128), jnp.float32),
    scratch_types=[
        pltpu.VMEM((CH,), jnp.int32),
        pltpu.VMEM((CH,), jnp.int32),
        pltpu.VMEM((CH, 16), jnp.float32),
        pltpu.VMEM((CH // 4, 16), jnp.float32),
        pltpu.SemaphoreType.DMA,
    ],
)
def _sc_gather(tok_hbm, turns4_hbm, out_hbm, tok_v, gran_v, rows_v, comp_v, sem):
    wid = lax.axis_index("s") * NC + lax.axis_index("c")
    base = wid * _BPW
    lane = lax.iota(jnp.int32, 16)
    crow_off = lane >> 2
    ccol_base = (lane & 3) * 4

    def chunk_body(i, carry):
        off = base + i * CH
        pltpu.sync_copy(tok_hbm.at[pl.ds(off, CH)], tok_v)

        def gran_body(g, c):
            t = tok_v[pl.ds(g * 16, 16)]
            gran_v[pl.ds(g * 16, 16)] = lax.shift_right_logical(t, 2)
            return c

        lax.fori_loop(0, CH // 16, gran_body, 0)

        pltpu.async_copy(turns4_hbm.at[gran_v], rows_v, sem).wait()

        def comp_body(g, c):
            tokv = tok_v[pl.ds(g * 16, 16)]
            sub = (tokv & 3) * 4
            rowi = g * 16 + lane
            crow = ((g >> 5) * 128) + ((g & 31) * 4) + crow_off
            for j in range(4):
                v = plsc.load_gather(rows_v, [rowi, sub + j])
                plsc.store_scatter(comp_v, [crow, ccol_base + j], v)
            return c

        lax.fori_loop(0, CH // 16, comp_body, 0)

        for u in range(_UPC):
            ug = off // _UNIT + u
            pltpu.sync_copy(
                comp_v.at[pl.ds(u * 128, 128)],
                out_hbm.at[pl.ds((ug // 8) * 128, 128), pl.ds((ug % 8) * 16, 16)],
            )
        return carry

    lax.fori_loop(0, _NCHUNK, chunk_body, 0)


TBR = 128
N4 = N // 4
N32 = N // 32


def _poly_body(tv_ref, w_ref, b_ref, o_ref):
    t = jnp.clip(jnp.trunc(tv_ref[...]), -128.0, 127.0)
    t2 = t * t
    t3 = t2 * t
    t4 = t2 * t2
    dot = lambda a, w: lax.dot_general(
        a, w, (((1,), (0,)), ((), ())), preferred_element_type=jnp.float32
    )
    for k in range(8):
        acc = b_ref[...] + dot(t[:, k * 16:(k + 1) * 16], w_ref[0])
        acc = acc + dot(t2[:, k * 16:(k + 1) * 16], w_ref[1])
        acc = acc + dot(t3[:, k * 16:(k + 1) * 16], w_ref[2])
        acc = acc + dot(t4[:, k * 16:(k + 1) * 16], w_ref[3])
        o_ref[pl.ds(k * TBR, TBR), :] = acc


def _tc_poly(tv3, Wd, bias2):
    grid = (N32 // TBR,)
    return pl.pallas_call(
        _poly_body,
        grid=grid,
        in_specs=[
            pl.BlockSpec((TBR, 128), lambda i: (i, 0)),
            pl.BlockSpec((NTURNS, 4 * NTURNS, 128), lambda i: (0, 0, 0)),
            pl.BlockSpec((1, 128), lambda i: (0, 0)),
        ],
        out_specs=pl.BlockSpec((TBR * 8, 128), lambda i: (i, 0)),
        out_shape=jax.ShapeDtypeStruct((N4, 128), jnp.float32),
    )(tv3, Wd, bias2)


def kernel(token_ids, turns, poly_coeffs):
    tok_flat = token_ids.reshape(N).astype(jnp.int32)
    turnsT = jnp.swapaxes(turns, 0, 1)
    turns4 = _sc_interleave(turnsT)
    tv3 = _sc_gather(tok_flat, turns4)

    eye4 = jnp.eye(4, dtype=jnp.float32)
    W = jnp.einsum("kK,tdo->dktKo", eye4, poly_coeffs)
    W = W.reshape(DEG + 1, 4 * NTURNS, 4 * OUTDIM)
    bias2 = W[0].sum(axis=0, keepdims=True)
    Wd = W[1:]

    out2 = _tc_poly(tv3, Wd, bias2)
    return out2.reshape(B, S, OUTDIM)


# speedup vs baseline: 1.9710x; 1.9710x over previous
import functools

import jax
import jax.numpy as jnp
from jax import lax
from jax.experimental import pallas as pl
from jax.experimental.pallas import tpu as pltpu
from jax.experimental.pallas import tpu_sc as plsc

VOCAB = 1000000
NTURNS = 4
OUTDIM = 32
DEG = 4
B = 4096
S = 200
N = B * S
NGRAN = VOCAB // 4

_info = plsc.get_sparse_core_info()
NC, NS = _info.num_cores, _info.num_subcores
NW = NC * NS

_sc_params = pltpu.CompilerParams(
    use_tc_tiling_on_sc=False, needs_layout_passes=False
)
_mesh = plsc.VectorSubcoreMesh(core_axis_name="c", subcore_axis_name="s")


_CG = 1956
_ICHUNKS = 4
_GW = _CG * _ICHUNKS
assert _CG * 4 % 16 == 0 and NW * _GW >= NGRAN and _GW <= NGRAN


@functools.partial(
    pl.kernel,
    mesh=_mesh,
    compiler_params=_sc_params,
    out_type=jax.ShapeDtypeStruct((NGRAN, 16), jnp.float32),
    scratch_types=[
        pltpu.VMEM((NTURNS, 4 * _CG), jnp.float32),
        pltpu.VMEM((_CG, 16), jnp.float32),
    ],
)
def _sc_interleave(turnsT_hbm, out_hbm, in_v, out_v):
    wid = lax.axis_index("s") * NC + lax.axis_index("c")
    gw0 = jnp.minimum(wid * _GW, NGRAN - _GW)
    lane = lax.iota(jnp.int32, 16)
    grow = lane >> 2
    gcol = (lane & 3) * 4

    def chunk(i, carry):
        g0 = gw0 + i * _CG
        v0 = g0 * 4
        for t in range(NTURNS):
            pltpu.sync_copy(turnsT_hbm.at[t, pl.ds(v0, 4 * _CG)], in_v.at[t])

        def body(u, c):
            for t in range(NTURNS):
                vals = in_v[t, pl.ds(u * 16, 16)]
                plsc.store_scatter(out_v, [(u * 4) + grow, gcol + t], vals)
            return c

        lax.fori_loop(0, 4 * _CG // 16, body, 0)
        pltpu.sync_copy(out_v, out_hbm.at[pl.ds(g0, _CG)])
        return carry

    lax.fori_loop(0, _ICHUNKS, chunk, 0)


CH = 3200
_BPW = N // NW
_NCHUNK = _BPW // CH


@functools.partial(
    pl.kernel,
    mesh=_mesh,
    compiler_params=_sc_params,
    out_type=jax.ShapeDtypeStruct((NTURNS * S, B), jnp.float32),
    scratch_types=[
        pltpu.VMEM((CH,), jnp.int32),
        pltpu.VMEM((CH,), jnp.int32),
        pltpu.VMEM((CH, 16), jnp.float32),
        pltpu.VMEM((NTURNS * S, 16), jnp.float32),
        pltpu.SemaphoreType.DMA,
    ],
)
def _sc_gather(tok_hbm, turns4_hbm, out_hbm, tok_v, gran_v, rows_v, comp_v, sem):
    wid = lax.axis_index("s") * NC + lax.axis_index("c")
    base = wid * _BPW
    lane = lax.iota(jnp.int32, 16)

    def chunk_body(i, carry):
        off = base + i * CH
        pltpu.sync_copy(tok_hbm.at[pl.ds(off, CH)], tok_v)

        def gran_body(g, c):
            t = tok_v[pl.ds(g * 16, 16)]
            gran_v[pl.ds(g * 16, 16)] = lax.shift_right_logical(t, 2)
            return c

        lax.fori_loop(0, CH // 16, gran_body, 0)

        pltpu.async_copy(turns4_hbm.at[gran_v], rows_v, sem).wait()

        def comp_body(g, c):
            tokv = tok_v[pl.ds(g * 16, 16)]
            sub = (tokv & 3) * 4
            rowi = g * 16 + lane
            bl = rowi // S
            crow0 = (rowi - bl * S) * 4
            for j in range(4):
                v = plsc.load_gather(rows_v, [rowi, sub + j])
                plsc.store_scatter(comp_v, [crow0 + j, bl], v)
            return c

        lax.fori_loop(0, CH // 16, comp_body, 0)

        pltpu.sync_copy(
            comp_v, out_hbm.at[:, pl.ds(wid * (_BPW // S) + i * (CH // S), CH // S)]
        )
        return carry

    lax.fori_loop(0, _NCHUNK, chunk_body, 0)


_SB = 8
_BB = 512


def _poly_body(tv_ref, w_ref, b_ref, o_ref):
    t1 = jnp.clip(jnp.trunc(tv_ref[...]), -128.0, 127.0)
    t2 = t1 * t1
    t3 = t2 * t1
    t4 = t2 * t2
    A = jnp.concatenate([t1, t2, t3, t4], axis=0)
    res = lax.dot_general(
        w_ref[...], A, (((1,), (0,)), ((), ())), preferred_element_type=jnp.float32
    )
    o_ref[...] = (res + b_ref[...]).reshape(_SB, OUTDIM, _BB)


def _tc_poly(tv_s, Wb, bias256):
    grid = (S // _SB, B // _BB)
    return pl.pallas_call(
        _poly_body,
        grid=grid,
        in_specs=[
            pl.BlockSpec((_SB * NTURNS, _BB), lambda i, j: (i, j)),
            pl.BlockSpec((_SB * OUTDIM, DEG * _SB * NTURNS), lambda i, j: (0, 0)),
            pl.BlockSpec((_SB * OUTDIM, 1), lambda i, j: (0, 0)),
        ],
        out_specs=pl.BlockSpec((_SB, OUTDIM, _BB), lambda i, j: (i, 0, j)),
        out_shape=jax.ShapeDtypeStruct((S, OUTDIM, B), jnp.float32),
    )(tv_s, Wb, bias256)


def kernel(token_ids, turns, poly_coeffs):
    tok_flat = token_ids.reshape(N).astype(jnp.int32)
    turnsT = jnp.swapaxes(turns, 0, 1)
    turns4 = _sc_interleave(turnsT)
    tv_s = _sc_gather(tok_flat, turns4)

    eye8 = jnp.eye(_SB, dtype=jnp.float32)
    Wb = jnp.einsum("sS,tdo->sodSt", eye8, poly_coeffs[:, 1:, :])
    Wb = Wb.reshape(_SB * OUTDIM, DEG * _SB * NTURNS)
    bias256 = jnp.tile(poly_coeffs[:, 0, :].sum(axis=0), _SB)[:, None]

    out_T = _tc_poly(tv_s, Wb, bias256)
    return jnp.transpose(out_T, (2, 0, 1))


# speedup vs baseline: 2.1218x; 1.0765x over previous
import functools

import jax
import jax.numpy as jnp
from jax import lax
from jax.experimental import pallas as pl
from jax.experimental.pallas import tpu as pltpu
from jax.experimental.pallas import tpu_sc as plsc

VOCAB = 1000000
NTURNS = 4
OUTDIM = 32
DEG = 4
B = 4096
S = 200
N = B * S
NGRAN = VOCAB // 4

_info = plsc.get_sparse_core_info()
NC, NS = _info.num_cores, _info.num_subcores
NW = NC * NS

_sc_params = pltpu.CompilerParams(
    use_tc_tiling_on_sc=False, needs_layout_passes=False
)
_mesh = plsc.VectorSubcoreMesh(core_axis_name="c", subcore_axis_name="s")


_CG = 1956
_ICHUNKS = 4
_GW = _CG * _ICHUNKS
assert _CG * 4 % 16 == 0 and NW * _GW >= NGRAN and _GW <= NGRAN


@functools.partial(
    pl.kernel,
    mesh=_mesh,
    compiler_params=_sc_params,
    out_type=jax.ShapeDtypeStruct((NGRAN, 16), jnp.float32),
    scratch_types=[
        pltpu.VMEM((NTURNS, 4 * _CG), jnp.float32),
        pltpu.VMEM((_CG, 16), jnp.float32),
    ],
)
def _sc_interleave(turnsT_hbm, out_hbm, in_v, out_v):
    wid = lax.axis_index("s") * NC + lax.axis_index("c")
    gw0 = jnp.minimum(wid * _GW, NGRAN - _GW)
    lane = lax.iota(jnp.int32, 16)
    grow = lane >> 2
    gcol = (lane & 3) * 4

    def chunk(i, carry):
        g0 = gw0 + i * _CG
        v0 = g0 * 4
        for t in range(NTURNS):
            pltpu.sync_copy(turnsT_hbm.at[t, pl.ds(v0, 4 * _CG)], in_v.at[t])

        def body(u, c):
            for t in range(NTURNS):
                vals = in_v[t, pl.ds(u * 16, 16)]
                plsc.store_scatter(out_v, [(u * 4) + grow, gcol + t], vals)
            return c

        lax.fori_loop(0, 4 * _CG // 16, body, 0)
        pltpu.sync_copy(out_v, out_hbm.at[pl.ds(g0, _CG)])
        return carry

    lax.fori_loop(0, _ICHUNKS, chunk, 0)


CH = 1600
_BPW = N // NW
_NCHUNK = _BPW // CH
_CPP = CH // S


@functools.partial(
    pl.kernel,
    mesh=_mesh,
    compiler_params=_sc_params,
    out_type=jax.ShapeDtypeStruct((NTURNS * S, B), jnp.float32),
    scratch_types=[
        pltpu.VMEM((CH,), jnp.int32),
        pltpu.VMEM((CH,), jnp.int32),
        pltpu.VMEM((CH, 16), jnp.float32),
        pltpu.VMEM((CH,), jnp.int32),
        pltpu.VMEM((CH,), jnp.int32),
        pltpu.VMEM((CH, 16), jnp.float32),
        pltpu.VMEM((NTURNS * S, 16), jnp.float32),
        pltpu.SemaphoreType.DMA,
        pltpu.SemaphoreType.DMA,
    ],
)
def _sc_gather(
    tok_hbm, turns4_hbm, out_hbm,
    tok0, gran0, rows0, tok1, gran1, rows1, comp_v, sem0, sem1,
):
    wid = lax.axis_index("s") * NC + lax.axis_index("c")
    base = wid * _BPW
    lane = lax.iota(jnp.int32, 16)

    def start(i, tok_v, gran_v, rows_v, sem):
        off = base + i * CH
        pltpu.sync_copy(tok_hbm.at[pl.ds(off, CH)], tok_v)

        def gran_body(g, c):
            t = tok_v[pl.ds(g * 16, 16)]
            gran_v[pl.ds(g * 16, 16)] = lax.shift_right_logical(t, 2)
            return c

        lax.fori_loop(0, CH // 16, gran_body, 0)
        pltpu.async_copy(turns4_hbm.at[gran_v], rows_v, sem)

    def wait(gran_v, rows_v, sem):
        pltpu.make_async_copy(turns4_hbm.at[gran_v], rows_v, sem).wait()

    def compact(tok_v, rows_v, colofs):
        def comp_body(g, c):
            tokv = tok_v[pl.ds(g * 16, 16)]
            sub = (tokv & 3) * 4
            rowi = g * 16 + lane
            bl = rowi // S
            crow0 = (rowi - bl * S) * 4
            for j in range(4):
                v = plsc.load_gather(rows_v, [rowi, sub + j])
                plsc.store_scatter(comp_v, [crow0 + j, bl + colofs], v)
            return c

        lax.fori_loop(0, CH // 16, comp_body, 0)

    start(0, tok0, gran0, rows0, sem0)

    def pair(p, carry):
        i0 = 2 * p
        start(i0 + 1, tok1, gran1, rows1, sem1)
        wait(gran0, rows0, sem0)
        compact(tok0, rows0, 0)

        @pl.when(i0 + 2 < _NCHUNK)
        def _():
            start(i0 + 2, tok0, gran0, rows0, sem0)

        wait(gran1, rows1, sem1)
        compact(tok1, rows1, _CPP)
        pltpu.sync_copy(
            comp_v,
            out_hbm.at[:, pl.ds(wid * (_BPW // S) + p * 2 * _CPP, 2 * _CPP)],
        )
        return carry

    lax.fori_loop(0, _NCHUNK // 2, pair, 0)


_SB = 8
_BB = 512


def _poly_body(tv_ref, w_ref, b_ref, o_ref):
    t1 = jnp.clip(jnp.trunc(tv_ref[...]), -128.0, 127.0)
    t2 = t1 * t1
    t3 = t2 * t1
    t4 = t2 * t2
    A = jnp.concatenate([t1, t2, t3, t4], axis=0)
    res = lax.dot_general(
        w_ref[...], A, (((1,), (0,)), ((), ())), preferred_element_type=jnp.float32
    )
    o_ref[...] = (res + b_ref[...]).reshape(_SB, OUTDIM, _BB)


def _tc_poly(tv_s, Wb, bias256):
    grid = (S // _SB, B // _BB)
    return pl.pallas_call(
        _poly_body,
        grid=grid,
        in_specs=[
            pl.BlockSpec((_SB * NTURNS, _BB), lambda i, j: (i, j)),
            pl.BlockSpec((_SB * OUTDIM, DEG * _SB * NTURNS), lambda i, j: (0, 0)),
            pl.BlockSpec((_SB * OUTDIM, 1), lambda i, j: (0, 0)),
        ],
        out_specs=pl.BlockSpec((_SB, OUTDIM, _BB), lambda i, j: (i, 0, j)),
        out_shape=jax.ShapeDtypeStruct((S, OUTDIM, B), jnp.float32),
    )(tv_s, Wb, bias256)


def kernel(token_ids, turns, poly_coeffs):
    tok_flat = token_ids.reshape(N).astype(jnp.int32)
    turnsT = jnp.swapaxes(turns, 0, 1)
    turns4 = _sc_interleave(turnsT)
    tv_s = _sc_gather(tok_flat, turns4)

    eye8 = jnp.eye(_SB, dtype=jnp.float32)
    Wb = jnp.einsum("sS,tdo->sodSt", eye8, poly_coeffs[:, 1:, :])
    Wb = Wb.reshape(_SB * OUTDIM, DEG * _SB * NTURNS)
    bias256 = jnp.tile(poly_coeffs[:, 0, :].sum(axis=0), _SB)[:, None]

    out_T = _tc_poly(tv_s, Wb, bias256)
    return jnp.transpose(out_T, (2, 0, 1))


# speedup vs baseline: 2.2795x; 1.0743x over previous
import functools

import jax
import jax.numpy as jnp
from jax import lax
from jax.experimental import pallas as pl
from jax.experimental.pallas import tpu as pltpu
from jax.experimental.pallas import tpu_sc as plsc

VOCAB = 1000000
NTURNS = 4
OUTDIM = 32
DEG = 4
B = 4096
S = 200
N = B * S
NGRAN = VOCAB // 4

_info = plsc.get_sparse_core_info()
NC, NS = _info.num_cores, _info.num_subcores
NW = NC * NS

_sc_params = pltpu.CompilerParams(
    use_tc_tiling_on_sc=False, needs_layout_passes=False
)
_mesh = plsc.VectorSubcoreMesh(core_axis_name="c", subcore_axis_name="s")


_CG = 1956
_ICHUNKS = 4
_GW = _CG * _ICHUNKS
assert _CG * 4 % 16 == 0 and NW * _GW >= NGRAN and _GW <= NGRAN


@functools.partial(
    pl.kernel,
    mesh=_mesh,
    compiler_params=_sc_params,
    out_type=jax.ShapeDtypeStruct((NGRAN, 16), jnp.float32),
    scratch_types=[
        pltpu.VMEM((NTURNS, 4 * _CG), jnp.float32),
        pltpu.VMEM((NTURNS, 4 * _CG), jnp.float32),
        pltpu.VMEM((_CG, 16), jnp.float32),
        pltpu.VMEM((_CG, 16), jnp.float32),
        pltpu.SemaphoreType.DMA,
        pltpu.SemaphoreType.DMA,
        pltpu.SemaphoreType.DMA,
        pltpu.SemaphoreType.DMA,
    ],
)
def _sc_interleave(turnsT_hbm, out_hbm, in0, in1, out0, out1, si0, si1, so0, so1):
    wid = lax.axis_index("s") * NC + lax.axis_index("c")
    gw0 = jnp.minimum(wid * _GW, NGRAN - _GW)
    lane = lax.iota(jnp.int32, 16)
    grow = lane >> 2
    gcol = (lane & 3) * 4
    ins, outs = (in0, in1), (out0, out1)
    sis, sos = (si0, si1), (so0, so1)

    def in_start(i, in_v, sem):
        v0 = (gw0 + i * _CG) * 4
        for t in range(NTURNS):
            pltpu.async_copy(turnsT_hbm.at[t, pl.ds(v0, 4 * _CG)], in_v.at[t], sem)

    def in_wait(i, in_v, sem):
        v0 = (gw0 + i * _CG) * 4
        for t in range(NTURNS):
            pltpu.make_async_copy(
                turnsT_hbm.at[t, pl.ds(v0, 4 * _CG)], in_v.at[t], sem
            ).wait()

    def out_wait(i, out_v, sem):
        g0 = gw0 + i * _CG
        pltpu.make_async_copy(out_v, out_hbm.at[pl.ds(g0, _CG)], sem).wait()

    in_start(0, ins[0], sis[0])
    for i in range(_ICHUNKS):
        if i + 1 < _ICHUNKS:
            in_start(i + 1, ins[(i + 1) % 2], sis[(i + 1) % 2])
        in_wait(i, ins[i % 2], sis[i % 2])
        if i >= 2:
            out_wait(i - 2, outs[i % 2], sos[i % 2])
        in_v, out_v = ins[i % 2], outs[i % 2]

        def body(u, c, in_v=in_v, out_v=out_v):
            for t in range(NTURNS):
                vals = in_v[t, pl.ds(u * 16, 16)]
                plsc.store_scatter(out_v, [(u * 4) + grow, gcol + t], vals)
            return c

        lax.fori_loop(0, 4 * _CG // 16, body, 0)
        pltpu.async_copy(out_v, out_hbm.at[pl.ds(gw0 + i * _CG, _CG)], sos[i % 2])
    out_wait(_ICHUNKS - 2, outs[_ICHUNKS % 2], sos[_ICHUNKS % 2])
    out_wait(_ICHUNKS - 1, outs[(_ICHUNKS - 1) % 2], sos[(_ICHUNKS - 1) % 2])


CH = 1600
_NBC = 4
BC = B // _NBC
_BPW = N // _NBC // NW
_NCHUNK = _BPW // CH
_CPP = CH // S


def _make_sc_gather(cbc):
  @functools.partial(
      pl.kernel,
      mesh=_mesh,
      compiler_params=_sc_params,
      out_type=jax.ShapeDtypeStruct((NTURNS * S, BC), jnp.float32),
      scratch_types=[
          pltpu.VMEM((CH,), jnp.int32),
          pltpu.VMEM((CH,), jnp.int32),
          pltpu.VMEM((CH, 16), jnp.float32),
          pltpu.VMEM((CH,), jnp.int32),
          pltpu.VMEM((CH,), jnp.int32),
          pltpu.VMEM((CH, 16), jnp.float32),
          pltpu.VMEM((NTURNS * S, 16), jnp.float32),
          pltpu.SemaphoreType.DMA,
          pltpu.SemaphoreType.DMA,
      ],
  )
  def _sc_gather(
      tok_hbm, turns4_hbm, prev_hbm, out_hbm,
      tok0, gran0, rows0, tok1, gran1, rows1, comp_v, sem0, sem1,
  ):
    del prev_hbm
    wid = lax.axis_index("s") * NC + lax.axis_index("c")
    base = cbc * (N // _NBC) + wid * _BPW
    lane = lax.iota(jnp.int32, 16)

    def start(i, tok_v, gran_v, rows_v, sem):
        off = base + i * CH
        pltpu.sync_copy(tok_hbm.at[pl.ds(off, CH)], tok_v)

        def gran_body(g, c):
            t = tok_v[pl.ds(g * 16, 16)]
            gran_v[pl.ds(g * 16, 16)] = lax.shift_right_logical(t, 2)
            return c

        lax.fori_loop(0, CH // 16, gran_body, 0)
        pltpu.async_copy(turns4_hbm.at[gran_v], rows_v, sem)

    def wait(gran_v, rows_v, sem):
        pltpu.make_async_copy(turns4_hbm.at[gran_v], rows_v, sem).wait()

    def compact(tok_v, rows_v, colofs):
        def comp_body(g, c):
            tokv = tok_v[pl.ds(g * 16, 16)]
            sub = (tokv & 3) * 4
            rowi = g * 16 + lane
            bl = rowi // S
            crow0 = (rowi - bl * S) * 4
            for j in range(4):
                v = plsc.load_gather(rows_v, [rowi, sub + j])
                plsc.store_scatter(comp_v, [crow0 + j, bl + colofs], v)
            return c

        lax.fori_loop(0, CH // 16, comp_body, 0)

    start(0, tok0, gran0, rows0, sem0)

    def pair(p, carry):
        i0 = 2 * p
        start(i0 + 1, tok1, gran1, rows1, sem1)
        wait(gran0, rows0, sem0)
        compact(tok0, rows0, 0)

        @pl.when(i0 + 2 < _NCHUNK)
        def _():
            start(i0 + 2, tok0, gran0, rows0, sem0)

        wait(gran1, rows1, sem1)
        compact(tok1, rows1, _CPP)
        pltpu.sync_copy(
            comp_v,
            out_hbm.at[:, pl.ds(wid * (_BPW // S) + p * 2 * _CPP, 2 * _CPP)],
        )
        return carry

    lax.fori_loop(0, _NCHUNK // 2, pair, 0)

  return _sc_gather


_sc_gathers = [_make_sc_gather(c) for c in range(_NBC)]


_SB = 8
_BB = 512


def _poly_body(tv_ref, w_ref, b_ref, o_ref):
    t1 = jnp.clip(jnp.trunc(tv_ref[...]), -128.0, 127.0)
    t2 = t1 * t1
    t3 = t2 * t1
    t4 = t2 * t2
    A = jnp.concatenate([t1, t2, t3, t4], axis=0)
    res = lax.dot_general(
        w_ref[...], A, (((1,), (0,)), ((), ())), preferred_element_type=jnp.float32
    )
    o_ref[...] = (res + b_ref[...]).reshape(_SB, OUTDIM, _BB)


def _poly_body_acc(cur_ref, tv_ref, w_ref, b_ref, o_ref):
    _poly_body(tv_ref, w_ref, b_ref, o_ref)


def _tc_poly_chunk(cbc, cur, tv_c, Wb, bias256):
    grid = (S // _SB, BC // _BB)
    out_spec = pl.BlockSpec(
        (_SB, OUTDIM, _BB), lambda i, j: (i, 0, j + cbc * (BC // _BB))
    )
    common = dict(
        grid=grid,
        out_specs=out_spec,
        out_shape=jax.ShapeDtypeStruct((S, OUTDIM, B), jnp.float32),
    )
    tv_spec = pl.BlockSpec((_SB * NTURNS, _BB), lambda i, j: (i, j))
    w_spec = pl.BlockSpec((_SB * OUTDIM, DEG * _SB * NTURNS), lambda i, j: (0, 0))
    b_spec = pl.BlockSpec((_SB * OUTDIM, 1), lambda i, j: (0, 0))
    if cur is None:
        return pl.pallas_call(
            _poly_body, in_specs=[tv_spec, w_spec, b_spec], **common
        )(tv_c, Wb, bias256)
    return pl.pallas_call(
        _poly_body_acc,
        in_specs=[pl.BlockSpec(memory_space=pltpu.HBM), tv_spec, w_spec, b_spec],
        input_output_aliases={0: 0},
        **common,
    )(cur, tv_c, Wb, bias256)


def kernel(token_ids, turns, poly_coeffs):
    tok_flat = token_ids.reshape(N).astype(jnp.int32)
    turnsT = jnp.swapaxes(turns, 0, 1)
    turns4 = _sc_interleave(turnsT)

    eye8 = jnp.eye(_SB, dtype=jnp.float32)
    Wb = jnp.einsum("sS,tdo->sodSt", eye8, poly_coeffs[:, 1:, :])
    Wb = Wb.reshape(_SB * OUTDIM, DEG * _SB * NTURNS)
    bias256 = jnp.tile(poly_coeffs[:, 0, :].sum(axis=0), _SB)[:, None]

    tvs = []
    prev = turns4
    for c in range(_NBC):
        prev = _sc_gathers[c](tok_flat, turns4, prev)
        tvs.append(prev)
    out_T = None
    for c in range(_NBC):
        out_T = _tc_poly_chunk(c, out_T, tvs[c], Wb, bias256)
    return jnp.transpose(out_T, (2, 0, 1))


# speedup vs baseline: 2.7009x; 1.1849x over previous
import functools

import jax
import jax.numpy as jnp
from jax import lax
from jax.experimental import pallas as pl
from jax.experimental.pallas import tpu as pltpu
from jax.experimental.pallas import tpu_sc as plsc

VOCAB = 1000000
NTURNS = 4
OUTDIM = 32
DEG = 4
B = 4096
S = 200
N = B * S
NGRAN = VOCAB // 4

_info = plsc.get_sparse_core_info()
NC, NS = _info.num_cores, _info.num_subcores
NW = NC * NS

_sc_params = pltpu.CompilerParams(
    use_tc_tiling_on_sc=False, needs_layout_passes=False
)
_mesh = plsc.VectorSubcoreMesh(core_axis_name="c", subcore_axis_name="s")


_CG = 1956
_ICHUNKS = 4
_GW = _CG * _ICHUNKS
assert _CG * 4 % 16 == 0 and NW * _GW >= NGRAN and _GW <= NGRAN


@functools.partial(
    pl.kernel,
    mesh=_mesh,
    compiler_params=_sc_params,
    out_type=jax.ShapeDtypeStruct((NGRAN, 16), jnp.float32),
    scratch_types=[
        pltpu.VMEM((NTURNS, 4 * _CG), jnp.float32),
        pltpu.VMEM((NTURNS, 4 * _CG), jnp.float32),
        pltpu.VMEM((_CG, 16), jnp.float32),
        pltpu.VMEM((_CG, 16), jnp.float32),
        pltpu.SemaphoreType.DMA,
        pltpu.SemaphoreType.DMA,
        pltpu.SemaphoreType.DMA,
        pltpu.SemaphoreType.DMA,
    ],
)
def _sc_interleave(turnsT_hbm, out_hbm, in0, in1, out0, out1, si0, si1, so0, so1):
    wid = lax.axis_index("s") * NC + lax.axis_index("c")
    gw0 = jnp.minimum(wid * _GW, NGRAN - _GW)
    lane = lax.iota(jnp.int32, 16)
    grow = lane >> 2
    gcol = (lane & 3) * 4
    ins, outs = (in0, in1), (out0, out1)
    sis, sos = (si0, si1), (so0, so1)

    def in_start(i, in_v, sem):
        v0 = (gw0 + i * _CG) * 4
        for t in range(NTURNS):
            pltpu.async_copy(turnsT_hbm.at[t, pl.ds(v0, 4 * _CG)], in_v.at[t], sem)

    def in_wait(i, in_v, sem):
        v0 = (gw0 + i * _CG) * 4
        for t in range(NTURNS):
            pltpu.make_async_copy(
                turnsT_hbm.at[t, pl.ds(v0, 4 * _CG)], in_v.at[t], sem
            ).wait()

    def out_wait(i, out_v, sem):
        g0 = gw0 + i * _CG
        pltpu.make_async_copy(out_v, out_hbm.at[pl.ds(g0, _CG)], sem).wait()

    in_start(0, ins[0], sis[0])
    for i in range(_ICHUNKS):
        if i + 1 < _ICHUNKS:
            in_start(i + 1, ins[(i + 1) % 2], sis[(i + 1) % 2])
        in_wait(i, ins[i % 2], sis[i % 2])
        if i >= 2:
            out_wait(i - 2, outs[i % 2], sos[i % 2])
        in_v, out_v = ins[i % 2], outs[i % 2]

        def body(u, c, in_v=in_v, out_v=out_v):
            for t in range(NTURNS):
                vals = in_v[t, pl.ds(u * 16, 16)]
                plsc.store_scatter(out_v, [(u * 4) + grow, gcol + t], vals)
            return c

        lax.fori_loop(0, 4 * _CG // 16, body, 0)
        pltpu.async_copy(out_v, out_hbm.at[pl.ds(gw0 + i * _CG, _CG)], sos[i % 2])
    out_wait(_ICHUNKS - 2, outs[_ICHUNKS % 2], sos[_ICHUNKS % 2])
    out_wait(_ICHUNKS - 1, outs[(_ICHUNKS - 1) % 2], sos[(_ICHUNKS - 1) % 2])


CH = 1600
_NBC = 4
BC = B // _NBC
_BPW = N // _NBC // NW
_NCHUNK = _BPW // CH
_CPP = CH // S


def _make_sc_gather(cbc):
  @functools.partial(
      pl.kernel,
      mesh=_mesh,
      compiler_params=_sc_params,
      out_type=jax.ShapeDtypeStruct((NTURNS * S, BC), jnp.float32),
      scratch_types=[
          pltpu.VMEM((CH,), jnp.int32),
          pltpu.VMEM((CH,), jnp.int32),
          pltpu.VMEM((CH, 16), jnp.float32),
          pltpu.VMEM((CH,), jnp.int32),
          pltpu.VMEM((CH,), jnp.int32),
          pltpu.VMEM((CH, 16), jnp.float32),
          pltpu.VMEM((NTURNS * S, 16), jnp.float32),
          pltpu.SemaphoreType.DMA,
          pltpu.SemaphoreType.DMA,
      ],
  )
  def _sc_gather(
      tok_hbm, turns4_hbm, prev_hbm, out_hbm,
      tok0, gran0, rows0, tok1, gran1, rows1, comp_v, sem0, sem1,
  ):
    del prev_hbm
    wid = lax.axis_index("s") * NC + lax.axis_index("c")
    base = cbc * (N // _NBC) + wid * _BPW
    lane = lax.iota(jnp.int32, 16)

    def start(i, tok_v, gran_v, rows_v, sem):
        off = base + i * CH
        pltpu.sync_copy(tok_hbm.at[pl.ds(off, CH)], tok_v)

        def gran_body(g, c):
            t = tok_v[pl.ds(g * 16, 16)]
            gran_v[pl.ds(g * 16, 16)] = lax.shift_right_logical(t, 2)
            return c

        lax.fori_loop(0, CH // 16, gran_body, 0)
        pltpu.async_copy(turns4_hbm.at[gran_v], rows_v, sem)

    def wait(gran_v, rows_v, sem):
        pltpu.make_async_copy(turns4_hbm.at[gran_v], rows_v, sem).wait()

    def compact(tok_v, rows_v, colofs):
        def comp_body(g, c):
            tokv = tok_v[pl.ds(g * 16, 16)]
            sub = (tokv & 3) * 4
            rowi = g * 16 + lane
            bl = rowi // S
            crow0 = (rowi - bl * S) * 4
            for j in range(4):
                v = plsc.load_gather(rows_v, [rowi, sub + j])
                plsc.store_scatter(comp_v, [crow0 + j, bl + colofs], v)
            return c

        lax.fori_loop(0, CH // 16, comp_body, 0)

    start(0, tok0, gran0, rows0, sem0)

    def pair(p, carry):
        i0 = 2 * p
        start(i0 + 1, tok1, gran1, rows1, sem1)
        wait(gran0, rows0, sem0)
        compact(tok0, rows0, 0)

        @pl.when(i0 + 2 < _NCHUNK)
        def _():
            start(i0 + 2, tok0, gran0, rows0, sem0)

        wait(gran1, rows1, sem1)
        compact(tok1, rows1, _CPP)
        pltpu.sync_copy(
            comp_v,
            out_hbm.at[:, pl.ds(wid * (_BPW // S) + p * 2 * _CPP, 2 * _CPP)],
        )
        return carry

    lax.fori_loop(0, _NCHUNK // 2, pair, 0)

  return _sc_gather


_sc_gathers = [_make_sc_gather(c) for c in range(_NBC)]


_SB = 8
_BB = 1024


def _poly_body(tv_ref, w_ref, b_ref, o_ref):
    t1 = jnp.clip(jnp.trunc(tv_ref[...]), -128.0, 127.0)
    t2 = t1 * t1
    t3 = t2 * t1
    t4 = t2 * t2
    A = jnp.concatenate([t1, t2, t3, t4], axis=0)
    res = lax.dot_general(
        w_ref[...], A, (((1,), (0,)), ((), ())), preferred_element_type=jnp.float32
    )
    o_ref[...] = (res + b_ref[...]).reshape(_SB, OUTDIM, _BB)


def _poly_body_acc(cur_ref, tv_ref, w_ref, b_ref, o_ref):
    _poly_body(tv_ref, w_ref, b_ref, o_ref)


def _tc_poly_chunk(cbc, cur, tv_c, Wb, bias256):
    grid = (S // _SB, BC // _BB)
    out_spec = pl.BlockSpec(
        (_SB, OUTDIM, _BB), lambda i, j: (i, 0, j + cbc * (BC // _BB))
    )
    common = dict(
        grid=grid,
        out_specs=out_spec,
        out_shape=jax.ShapeDtypeStruct((S, OUTDIM, B), jnp.float32),
    )
    tv_spec = pl.BlockSpec((_SB * NTURNS, _BB), lambda i, j: (i, j))
    w_spec = pl.BlockSpec((_SB * OUTDIM, DEG * _SB * NTURNS), lambda i, j: (0, 0))
    b_spec = pl.BlockSpec((_SB * OUTDIM, 1), lambda i, j: (0, 0))
    if cur is None:
        return pl.pallas_call(
            _poly_body, in_specs=[tv_spec, w_spec, b_spec], **common
        )(tv_c, Wb, bias256)
    return pl.pallas_call(
        _poly_body_acc,
        in_specs=[pl.BlockSpec(memory_space=pltpu.HBM), tv_spec, w_spec, b_spec],
        input_output_aliases={0: 0},
        **common,
    )(cur, tv_c, Wb, bias256)


def kernel(token_ids, turns, poly_coeffs):
    tok_flat = token_ids.reshape(N).astype(jnp.int32)
    turnsT = jnp.swapaxes(turns, 0, 1)
    turns4 = _sc_interleave(turnsT)

    eye8 = jnp.eye(_SB, dtype=jnp.float32)
    Wb = jnp.einsum("sS,tdo->sodSt", eye8, poly_coeffs[:, 1:, :])
    Wb = Wb.reshape(_SB * OUTDIM, DEG * _SB * NTURNS)
    bias256 = jnp.tile(poly_coeffs[:, 0, :].sum(axis=0), _SB)[:, None]

    tvs = []
    prev = turns4
    for c in range(_NBC):
        prev = _sc_gathers[c](tok_flat, turns4, prev)
        tvs.append(prev)
    out_T = None
    for c in range(_NBC):
        out_T = _tc_poly_chunk(c, out_T, tvs[c], Wb, bias256)
    return jnp.transpose(out_T, (2, 0, 1))
